# baseline (speedup 1.0000x reference)
import functools

import jax
import jax.numpy as jnp
from jax import lax
from jax.experimental import pallas as pl
from jax.experimental.pallas import tpu as pltpu
from jax.experimental.pallas import tpu_sc as plsc

_B, _M, _N, _C = 4, 2048, 4096, 128
_NC, _H1, _H2, _OUT = 20, 128, 256, 128
_TN = 512
_D = 128
_CH = 128


def _dot(a, b):
    return jnp.dot(a.astype(jnp.bfloat16), b.astype(jnp.bfloat16),
                   preferred_element_type=jnp.float32)


def _semantic_body(f_ref, w1_ref, b1_ref, w2_ref, b2_ref, lg_ref, p_ref):
    f = f_ref[0]
    h = jnp.maximum(_dot(f, w1_ref[...]) + b1_ref[...], 0.0)
    lg = _dot(h, w2_ref[...]) + b2_ref[...]
    lg_ref[0] = lg
    e = jnp.exp(lg - jnp.max(lg, axis=-1, keepdims=True))
    p_ref[0] = e / jnp.sum(e, axis=-1, keepdims=True)


def _select_body(tgt_ref, src_ref, p_ref, i0_ref, i1_ref, i2_ref,
                 w0_ref, w1_ref, w2_ref, sw_ref):
    t = tgt_ref[0]
    s = src_ref[0]
    t2 = jnp.sum(t * t, axis=1, keepdims=True)
    s2 = jnp.sum(s * s, axis=0, keepdims=True) * 0.25
    cross = _dot(t, s)
    sq = (t2 + s2) + cross

    inf = jnp.float32(jnp.inf)
    iota = lax.broadcasted_iota(jnp.int32, sq.shape, 1)

    vmin0 = jnp.min(sq, axis=1, keepdims=True)
    gt0 = sq > vmin0
    vmin1 = jnp.min(jnp.where(gt0, sq, inf), axis=1, keepdims=True)
    gt1 = sq > vmin1
    vmin2 = jnp.min(jnp.where(gt1, sq, inf), axis=1, keepdims=True)
    le2 = sq <= vmin2
    total = jnp.sum(le2.astype(jnp.float32))

    one = jnp.float32(1.0)
    zero = jnp.float32(0.0)

    def _fast(_):
        i0 = jnp.min(jnp.where(gt0, _M, iota), axis=1, keepdims=True)
        i1 = jnp.min(jnp.where(gt1 | ~gt0, _M, iota), axis=1, keepdims=True)
        i2 = jnp.min(jnp.where(le2 & gt1, iota, _M), axis=1, keepdims=True)
        asum = jnp.where(le2, one, zero)
        return i0, i1, i2, vmin1, vmin2, _dot(asum, p_ref[0])

    def _exact(_):
        i0 = jnp.min(jnp.where(sq == vmin0, iota, _M), axis=1, keepdims=True)
        m0 = iota == i0
        v1 = jnp.min(jnp.where(m0, inf, sq), axis=1, keepdims=True)
        i1 = jnp.min(jnp.where((sq == v1) & ~m0, iota, _M),
                     axis=1, keepdims=True)
        m1 = iota == i1
        m01 = m0 | m1
        v2 = jnp.min(jnp.where(m01, inf, sq), axis=1, keepdims=True)
        i2 = jnp.min(jnp.where((sq == v2) & ~m01, iota, _M),
                     axis=1, keepdims=True)
        asum = jnp.where(m01 | (iota == i2), one, zero)
        return i0, i1, i2, v1, v2, _dot(asum, p_ref[0])

    i0, i1, i2, v1, v2, sw3 = lax.cond(
        total == jnp.float32(3 * _TN), _fast, _exact, None)

    d0 = jnp.sqrt(jnp.maximum(vmin0, 1e-12))
    d1 = jnp.sqrt(jnp.maximum(v1, 1e-12))
    d2 = jnp.sqrt(jnp.maximum(v2, 1e-12))
    mx = jnp.maximum(-d0, jnp.maximum(-d1, -d2))
    e0 = jnp.exp(-d0 - mx)
    e1 = jnp.exp(-d1 - mx)
    e2 = jnp.exp(-d2 - mx)
    z = e0 + e1 + e2
    i0_ref[0, 0] = i0
    i1_ref[0, 0] = i1
    i2_ref[0, 0] = i2
    w0_ref[0, 0] = e0 / z
    w1_ref[0, 0] = e1 / z
    w2_ref[0, 0] = e2 / z
    sw_ref[0, 0] = sw3 * jnp.float32(1.0 / 3.0)


def _make_sc_gather(n_rows):
    info = plsc.get_sparse_core_info()
    nw = info.num_cores * info.num_subcores
    per_w = n_rows // nw
    n_chunks = per_w // _CH
    mesh = plsc.VectorSubcoreMesh(core_axis_name="c", subcore_axis_name="s")

    @functools.partial(
        pl.kernel, mesh=mesh,
        out_type=jax.ShapeDtypeStruct((n_rows, _D), jnp.float32),
        scratch_types=[
            pltpu.VMEM((_CH,), jnp.int32),
            pltpu.VMEM((_CH, _D), jnp.float32),
            pltpu.SemaphoreType.DMA,
        ],
    )
    def gather(table_hbm, idx_hbm, out_hbm, idx_v, rows_v, sem):
        wid = lax.axis_index("s") * info.num_cores + lax.axis_index("c")
        base = wid * per_w
        for c in range(n_chunks):
            off = base + c * _CH
            pltpu.sync_copy(idx_hbm.at[pl.ds(off, _CH)], idx_v)
            pltpu.async_copy(table_hbm.at[idx_v], rows_v, sem).wait()
            pltpu.sync_copy(rows_v, out_hbm.at[pl.ds(off, _CH)])

    return gather


def _combine_body(g_ref, w0_ref, w1_ref, w2_ref, sw_ref, u1a_ref, u1b_ref,
                  bu1_ref, u2_ref, bu2_ref, out_ref):
    f0 = g_ref[:, 0, :]
    f1 = g_ref[:, 1, :]
    f2 = g_ref[:, 2, :]
    w0, w1, w2 = w0_ref[0], w1_ref[0], w2_ref[0]

    def _bf(x):
        return x.astype(jnp.bfloat16).astype(jnp.float32)

    fused = (_bf(w0) * _bf(f0) + _bf(w1) * _bf(f1) + _bf(w2) * _bf(f2))
    semw = sw_ref[0]

    h = jnp.maximum(
        _dot(fused, u1a_ref[...]) + _dot(semw, u1b_ref[...])
        + bu1_ref[...], 0.0)
    out_ref[...] = _dot(h, u2_ref[...]) + bu2_ref[...]


def kernel(src_points, tgt_points, src_features, W1, b1, W2, b2,
           U1, bu1, U2, bu2):
    B, M, _ = src_points.shape
    N = tgt_points.shape[1]
    C = src_features.shape[-1]
    NC = W2.shape[-1]
    H2 = U1.shape[-1]
    OUT = U2.shape[-1]
    nblk = N // _TN

    logits, p = pl.pallas_call(
        _semantic_body,
        grid=(B,),
        in_specs=[
            pl.BlockSpec((1, M, C), lambda b: (b, 0, 0)),
            pl.BlockSpec((C, _H1), lambda b: (0, 0)),
            pl.BlockSpec((1, _H1), lambda b: (0, 0)),
            pl.BlockSpec((_H1, NC), lambda b: (0, 0)),
            pl.BlockSpec((1, NC), lambda b: (0, 0)),
        ],
        out_specs=[
            pl.BlockSpec((1, M, NC), lambda b: (b, 0, 0)),
            pl.BlockSpec((1, M, NC), lambda b: (b, 0, 0)),
        ],
        out_shape=[
            jax.ShapeDtypeStruct((B, M, NC), jnp.float32),
            jax.ShapeDtypeStruct((B, M, NC), jnp.float32),
        ],
    )(src_features, W1, b1.reshape(1, -1), W2, b2.reshape(1, -1))

    tgt_pad = jnp.concatenate(
        [tgt_points, jnp.zeros((B, N, 5), jnp.float32)], axis=2)
    src_pad = jnp.concatenate(
        [src_points.transpose(0, 2, 1) * -2.0,
         jnp.zeros((B, 5, M), jnp.float32)], axis=1)

    io_spec = pl.BlockSpec((1, 1, _TN, 1), lambda b, j: (b, j, 0, 0))
    io_shape = jax.ShapeDtypeStruct((B, nblk, _TN, 1), jnp.int32)
    wf_shape = jax.ShapeDtypeStruct((B, nblk, _TN, 1), jnp.float32)
    sw_spec = pl.BlockSpec((1, 1, _TN, NC), lambda b, j: (b, j, 0, 0))
    sw_shape = jax.ShapeDtypeStruct((B, nblk, _TN, NC), jnp.float32)
    i0, i1, i2, w0, w1, w2, sw = pl.pallas_call(
        _select_body,
        grid=(B, nblk),
        in_specs=[
            pl.BlockSpec((1, _TN, 8), lambda b, j: (b, j, 0)),
            pl.BlockSpec((1, 8, M), lambda b, j: (b, 0, 0)),
            pl.BlockSpec((1, M, NC), lambda b, j: (b, 0, 0)),
        ],
        out_specs=[io_spec] * 3 + [io_spec] * 3 + [sw_spec],
        out_shape=[io_shape, io_shape, io_shape, wf_shape, wf_shape, wf_shape,
                   sw_shape],
    )(tgt_pad, src_pad, p)

    offs = (jnp.arange(B, dtype=jnp.int32) * M)[:, None, None, None]
    idx_all = jnp.concatenate([i0 + offs, i1 + offs, i2 + offs], axis=3)
    gidx = idx_all.reshape(-1)

    table = src_features.reshape(B * M, _D)
    rows = _make_sc_gather(B * N * 3)(table, gidx)

    g = rows.reshape(B * N, 3, _D)
    wr = (B * nblk, _TN, 1)
    upsampled = pl.pallas_call(
        _combine_body,
        grid=(B * nblk,),
        in_specs=[
            pl.BlockSpec((_TN, 3, _D), lambda i: (i, 0, 0)),
            pl.BlockSpec((1, _TN, 1), lambda i: (i, 0, 0)),
            pl.BlockSpec((1, _TN, 1), lambda i: (i, 0, 0)),
            pl.BlockSpec((1, _TN, 1), lambda i: (i, 0, 0)),
            pl.BlockSpec((1, _TN, NC), lambda i: (i, 0, 0)),
            pl.BlockSpec((C, H2), lambda i: (0, 0)),
            pl.BlockSpec((NC, H2), lambda i: (0, 0)),
            pl.BlockSpec((1, H2), lambda i: (0, 0)),
            pl.BlockSpec((H2, OUT), lambda i: (0, 0)),
            pl.BlockSpec((1, OUT), lambda i: (0, 0)),
        ],
        out_specs=pl.BlockSpec((_TN, OUT), lambda i: (i, 0)),
        out_shape=jax.ShapeDtypeStruct((B * N, OUT), jnp.float32),
    )(g, w0.reshape(wr), w1.reshape(wr), w2.reshape(wr),
      sw.reshape(B * nblk, _TN, NC),
      U1[:C], U1[C:], bu1.reshape(1, -1), U2, bu2.reshape(1, -1))

    return (upsampled.reshape(B, N, OUT), logits)


# speedup vs baseline: 2.3415x; 2.3415x over previous
import jax
import jax.numpy as jnp
from jax.experimental import pallas as pl

_B, _M, _N, _C = 4, 2048, 4096, 128
_NC, _H1, _H2, _OUT = 20, 128, 256, 128
_TN = 1024

_HIGH = jax.lax.Precision.HIGHEST


def _dot(a, b):
    return jnp.dot(a.astype(jnp.bfloat16), b.astype(jnp.bfloat16),
                   preferred_element_type=jnp.float32)


def _semantic_body(f_ref, w1_ref, b1_ref, w2_ref, b2_ref, lg_ref, p_ref):
    f = f_ref[0]
    h = jnp.maximum(_dot(f, w1_ref[...]) + b1_ref[...], 0.0)
    lg = _dot(h, w2_ref[...]) + b2_ref[...]
    lg_ref[0] = lg
    e = jnp.exp(lg - jnp.max(lg, axis=-1, keepdims=True))
    p_ref[0] = e / jnp.sum(e, axis=-1, keepdims=True)


def _fuse_body(tgt_ref, src_ref, f_ref, p_ref, u1a_ref, u1b_ref, bu1_ref,
               u2_ref, bu2_ref, out_ref):
    t = tgt_ref[0]
    s = src_ref[0]
    t2 = jnp.sum(t * t, axis=1, keepdims=True)
    s2 = jnp.sum(s * s, axis=0, keepdims=True) * 0.25
    cross = _dot(t, s)
    sq = (t2 + s2) + cross

    inf = jnp.float32(jnp.inf)
    zero = jnp.float32(0.0)
    one = jnp.float32(1.0)

    def _weights(v0, v1, v2):
        d0 = jnp.sqrt(jnp.maximum(v0, 1e-12))
        d1 = jnp.sqrt(jnp.maximum(v1, 1e-12))
        d2 = jnp.sqrt(jnp.maximum(v2, 1e-12))
        nd0, nd1, nd2 = -d0, -d1, -d2
        mx = jnp.maximum(nd0, jnp.maximum(nd1, nd2))
        e0 = jnp.exp(nd0 - mx)
        e1 = jnp.exp(nd1 - mx)
        e2 = jnp.exp(nd2 - mx)
        z = e0 + e1 + e2
        return e0 / z, e1 / z, e2 / z

    vmin0 = jnp.min(sq, axis=1, keepdims=True)
    gt0 = sq > vmin0
    vmin1 = jnp.min(jnp.where(gt0, sq, inf), axis=1, keepdims=True)
    gt1 = sq > vmin1
    vmin2 = jnp.min(jnp.where(gt1, sq, inf), axis=1, keepdims=True)
    le2 = sq <= vmin2
    total = jnp.sum(le2.astype(jnp.float32))
    w0, w1, w2 = _weights(vmin0, vmin1, vmin2)
    wsum_fast = jnp.where(gt0, jnp.where(gt1, jnp.where(le2, w2, zero), w1),
                          w0)
    asum_fast = jnp.where(le2, one, zero)

    fused_fast = _dot(wsum_fast, f_ref[0])
    semw_fast = _dot(asum_fast, p_ref[0])

    def _exact_path(_):
        iota = jax.lax.broadcasted_iota(jnp.int32, sq.shape, 1)
        i0 = jnp.min(jnp.where(sq == vmin0, iota, _M), axis=1, keepdims=True)
        m0 = iota == i0
        v1 = jnp.min(jnp.where(m0, inf, sq), axis=1, keepdims=True)
        i1 = jnp.min(jnp.where((sq == v1) & ~m0, iota, _M),
                     axis=1, keepdims=True)
        m01 = m0 | (iota == i1)
        v2 = jnp.min(jnp.where(m01, inf, sq), axis=1, keepdims=True)
        i2 = jnp.min(jnp.where((sq == v2) & ~m01, iota, _M),
                     axis=1, keepdims=True)
        x0, x1, x2 = _weights(vmin0, v1, v2)
        ws = (jnp.where(iota == i0, x0, zero)
              + jnp.where(iota == i1, x1, zero)
              + jnp.where(iota == i2, x2, zero))
        asm = jnp.where(ws > zero, one, zero)
        return _dot(ws, f_ref[0]), _dot(asm, p_ref[0])

    fused, semw3 = jax.lax.cond(
        total == jnp.float32(3 * _TN), lambda _: (fused_fast, semw_fast),
        _exact_path, None)
    semw = semw3 * jnp.float32(1.0 / 3.0)

    h = jnp.maximum(
        _dot(fused, u1a_ref[...]) + _dot(semw, u1b_ref[...])
        + bu1_ref[...], 0.0)
    out_ref[0] = _dot(h, u2_ref[...]) + bu2_ref[...]


def kernel(src_points, tgt_points, src_features, W1, b1, W2, b2,
           U1, bu1, U2, bu2):
    B, M, _ = src_points.shape
    N = tgt_points.shape[1]
    C = src_features.shape[-1]
    NC = W2.shape[-1]
    H2 = U1.shape[-1]
    OUT = U2.shape[-1]

    logits, p = pl.pallas_call(
        _semantic_body,
        grid=(B,),
        in_specs=[
            pl.BlockSpec((1, M, C), lambda b: (b, 0, 0)),
            pl.BlockSpec((C, _H1), lambda b: (0, 0)),
            pl.BlockSpec((1, _H1), lambda b: (0, 0)),
            pl.BlockSpec((_H1, NC), lambda b: (0, 0)),
            pl.BlockSpec((1, NC), lambda b: (0, 0)),
        ],
        out_specs=[
            pl.BlockSpec((1, M, NC), lambda b: (b, 0, 0)),
            pl.BlockSpec((1, M, NC), lambda b: (b, 0, 0)),
        ],
        out_shape=[
            jax.ShapeDtypeStruct((B, M, NC), jnp.float32),
            jax.ShapeDtypeStruct((B, M, NC), jnp.float32),
        ],
    )(src_features, W1, b1.reshape(1, -1), W2, b2.reshape(1, -1))

    tgt_pad = jnp.concatenate(
        [tgt_points, jnp.zeros((B, N, 5), jnp.float32)], axis=2)
    src_pad = jnp.concatenate(
        [src_points.transpose(0, 2, 1) * -2.0,
         jnp.zeros((B, 5, M), jnp.float32)], axis=1)

    upsampled = pl.pallas_call(
        _fuse_body,
        grid=(B, N // _TN),
        in_specs=[
            pl.BlockSpec((1, _TN, 8), lambda b, j: (b, j, 0)),
            pl.BlockSpec((1, 8, M), lambda b, j: (b, 0, 0)),
            pl.BlockSpec((1, M, C), lambda b, j: (b, 0, 0)),
            pl.BlockSpec((1, M, NC), lambda b, j: (b, 0, 0)),
            pl.BlockSpec((C, H2), lambda b, j: (0, 0)),
            pl.BlockSpec((NC, H2), lambda b, j: (0, 0)),
            pl.BlockSpec((1, H2), lambda b, j: (0, 0)),
            pl.BlockSpec((H2, OUT), lambda b, j: (0, 0)),
            pl.BlockSpec((1, OUT), lambda b, j: (0, 0)),
        ],
        out_specs=pl.BlockSpec((1, _TN, OUT), lambda b, j: (b, j, 0)),
        out_shape=jax.ShapeDtypeStruct((B, N, OUT), jnp.float32),
    )(tgt_pad, src_pad, src_features, p, U1[:C], U1[C:],
      bu1.reshape(1, -1), U2, bu2.reshape(1, -1))

    return (upsampled, logits)
